# baseline (speedup 1.0000x reference)
import jax
import jax.numpy as jnp
from jax.experimental import pallas as pl
from jax.experimental.pallas import tpu as pltpu


def _shift_up(v, zero):
    return jnp.concatenate([zero, v[:, :-1]], axis=1)


def _shift_dn(v, zero):
    return jnp.concatenate([v[:, 1:], zero], axis=1)


def _scan_kernel(x_ref, b_ref, g1_ref, g2_ref, g3_ref, o_ref,
                 xs, bs, g1s, g2s, g3s, os):
    bcb, H, W = x_ref.shape

    xs[...] = jnp.transpose(x_ref[...], (2, 0, 1))
    bs[...] = jnp.transpose(b_ref[...], (2, 0, 1))
    g1s[...] = jnp.transpose(g1_ref[...], (2, 0, 1))
    g2s[...] = jnp.transpose(g2_ref[...], (2, 0, 1))
    g3s[...] = jnp.transpose(g3_ref[...], (2, 0, 1))

    zero = jnp.zeros((bcb, 1), jnp.float32)

    def step(w, carry):
        c_uu, c_u, c, c_d, c_dd = carry
        x = xs[w]
        b = bs[w]
        g1 = g1s[w]
        g2 = g2s[w]
        g3 = g3s[w]
        bx = b * x
        bxu = _shift_up(bx, zero)
        g1u = _shift_up(g1, zero)
        g2u = _shift_up(g2, zero)
        g3u = _shift_up(g3, zero)
        bxd = _shift_dn(bx, zero)
        g1d = _shift_dn(g1, zero)
        g2d = _shift_dn(g2, zero)
        g3d = _shift_dn(g3, zero)
        c_new = bx + g1 * c_u + g2 * c + g3 * c_d
        cu_new = bxu + g1u * c_uu + g2u * c_u + g3u * c
        cd_new = bxd + g1d * c + g2d * c_d + g3d * c_dd
        cuu_new = _shift_up(cu_new, zero)
        cdd_new = _shift_dn(cd_new, zero)
        os[w] = c_new
        return (cuu_new, cu_new, c_new, cd_new, cdd_new)

    z = jnp.zeros((bcb, H), jnp.float32)
    jax.lax.fori_loop(0, W, step, (z, z, z, z, z))
    o_ref[...] = jnp.transpose(os[...], (1, 2, 0))


def kernel(X, B, G1, G2, G3):
    Bsz, C, H, W = X.shape
    BC = Bsz * C
    bcb = min(32, BC)

    ins = [t.reshape(BC, H, W) for t in (X, B, G1, G2, G3)]

    spec = pl.BlockSpec((bcb, H, W), lambda i: (i, 0, 0))
    scratch = [pltpu.VMEM((W, bcb, H), jnp.float32) for _ in range(6)]
    out = pl.pallas_call(
        _scan_kernel,
        grid=(BC // bcb,),
        in_specs=[spec] * 5,
        out_specs=spec,
        out_shape=jax.ShapeDtypeStruct((BC, H, W), jnp.float32),
        scratch_shapes=scratch,
        compiler_params=pltpu.CompilerParams(
            dimension_semantics=("parallel",),
            vmem_limit_bytes=100 * 1024 * 1024,
        ),
    )(*ins)
    return out.reshape(Bsz, C, H, W)


# speedup vs baseline: 1.2234x; 1.2234x over previous
import jax
import jax.numpy as jnp
from jax.experimental import pallas as pl
from jax.experimental.pallas import tpu as pltpu


def _scan_kernel(x_ref, b_ref, g1_ref, g2_ref, g3_ref, o_ref,
                 t1_bx, t1_g1, t1_g2, t1_g3,
                 t2_bx, t2_g1, t2_g2, t2_g3,
                 os2, os1, sem):
    bcb, H, W = x_ref.shape

    t1_bx[...] = jnp.swapaxes(x_ref[...] * b_ref[...], 1, 2)
    t1_g1[...] = jnp.swapaxes(g1_ref[...], 1, 2)
    t1_g2[...] = jnp.swapaxes(g2_ref[...], 1, 2)
    t1_g3[...] = jnp.swapaxes(g3_ref[...], 1, 2)

    copies = []
    for src, dst in ((t1_bx, t2_bx), (t1_g1, t2_g1),
                     (t1_g2, t2_g2), (t1_g3, t2_g3)):
        for j in range(bcb):
            copies.append(pltpu.make_async_copy(src.at[j], dst.at[:, j], sem))
    for c in copies:
        c.start()
    for c in copies:
        c.wait()

    def step(w, prev):
        bx = t2_bx[w]
        g1 = t2_g1[w]
        g2 = t2_g2[w]
        g3 = t2_g3[w]
        zero = jnp.zeros((bcb, 1), jnp.float32)
        up = jnp.concatenate([zero, prev[:, :-1]], axis=1)
        dn = jnp.concatenate([prev[:, 1:], zero], axis=1)
        h = bx + g1 * up + g2 * prev + g3 * dn
        os2[w] = h
        return h

    jax.lax.fori_loop(0, W, step, jnp.zeros((bcb, H), jnp.float32))

    out_copies = [pltpu.make_async_copy(os2.at[:, j], os1.at[j], sem)
                  for j in range(bcb)]
    for c in out_copies:
        c.start()
    for c in out_copies:
        c.wait()
    o_ref[...] = jnp.swapaxes(os1[...], 1, 2)


def kernel(X, B, G1, G2, G3):
    Bsz, C, H, W = X.shape
    BC = Bsz * C
    bcb = min(32, BC)

    ins = [t.reshape(BC, H, W) for t in (X, B, G1, G2, G3)]

    spec = pl.BlockSpec((bcb, H, W), lambda i: (i, 0, 0))
    scratch = [pltpu.VMEM((bcb, W, H), jnp.float32) for _ in range(4)]
    scratch += [pltpu.VMEM((W, bcb, H), jnp.float32) for _ in range(4)]
    scratch.append(pltpu.VMEM((W, bcb, H), jnp.float32))
    scratch.append(pltpu.VMEM((bcb, W, H), jnp.float32))
    scratch.append(pltpu.SemaphoreType.DMA)
    out = pl.pallas_call(
        _scan_kernel,
        grid=(BC // bcb,),
        in_specs=[spec] * 5,
        out_specs=spec,
        out_shape=jax.ShapeDtypeStruct((BC, H, W), jnp.float32),
        scratch_shapes=scratch,
        compiler_params=pltpu.CompilerParams(
            dimension_semantics=("parallel",),
            vmem_limit_bytes=100 * 1024 * 1024,
        ),
    )(*ins)
    return out.reshape(Bsz, C, H, W)


# speedup vs baseline: 1.2247x; 1.0010x over previous
import jax
import jax.numpy as jnp
from jax.experimental import pallas as pl
from jax.experimental.pallas import tpu as pltpu


def _scan_kernel(x_ref, b_ref, g1_ref, g2_ref, g3_ref, o_ref,
                 t1, t2, os2, os1, sem):
    bcb, H, W = x_ref.shape

    t1[:, 0 * W:1 * W] = jnp.swapaxes(x_ref[...] * b_ref[...], 1, 2)
    t1[:, 1 * W:2 * W] = jnp.swapaxes(g1_ref[...], 1, 2)
    t1[:, 2 * W:3 * W] = jnp.swapaxes(g2_ref[...], 1, 2)
    t1[:, 3 * W:4 * W] = jnp.swapaxes(g3_ref[...], 1, 2)

    copies = [pltpu.make_async_copy(t1.at[j], t2.at[:, j], sem)
              for j in range(bcb)]
    for c in copies:
        c.start()
    for c in copies:
        c.wait()

    def step(w, prev):
        bx = t2[w]
        g1 = t2[W + w]
        g2 = t2[2 * W + w]
        g3 = t2[3 * W + w]
        zero = jnp.zeros((bcb, 1), jnp.float32)
        up = jnp.concatenate([zero, prev[:, :-1]], axis=1)
        dn = jnp.concatenate([prev[:, 1:], zero], axis=1)
        h = bx + g1 * up + g2 * prev + g3 * dn
        os2[w] = h
        return h

    jax.lax.fori_loop(0, W, step, jnp.zeros((bcb, H), jnp.float32))

    out_copies = [pltpu.make_async_copy(os2.at[:, j], os1.at[j], sem)
                  for j in range(bcb)]
    for c in out_copies:
        c.start()
    for c in out_copies:
        c.wait()
    o_ref[...] = jnp.swapaxes(os1[...], 1, 2)


def kernel(X, B, G1, G2, G3):
    Bsz, C, H, W = X.shape
    BC = Bsz * C
    bcb = min(32, BC)

    ins = [t.reshape(BC, H, W) for t in (X, B, G1, G2, G3)]

    spec = pl.BlockSpec((bcb, H, W), lambda i: (i, 0, 0))
    scratch = [
        pltpu.VMEM((bcb, 4 * W, H), jnp.float32),
        pltpu.VMEM((4 * W, bcb, H), jnp.float32),
        pltpu.VMEM((W, bcb, H), jnp.float32),
        pltpu.VMEM((bcb, W, H), jnp.float32),
        pltpu.SemaphoreType.DMA,
    ]
    out = pl.pallas_call(
        _scan_kernel,
        grid=(BC // bcb,),
        in_specs=[spec] * 5,
        out_specs=spec,
        out_shape=jax.ShapeDtypeStruct((BC, H, W), jnp.float32),
        scratch_shapes=scratch,
        compiler_params=pltpu.CompilerParams(
            dimension_semantics=("parallel",),
            vmem_limit_bytes=100 * 1024 * 1024,
        ),
    )(*ins)
    return out.reshape(Bsz, C, H, W)
